# baseline (speedup 1.0000x reference)
import jax
import jax.numpy as jnp
from jax.experimental import pallas as pl
from jax.experimental.pallas import tpu as pltpu


def _copy_body(emb_ref, out_ref, sem):
    seq = out_ref.shape[0]
    copy = pltpu.make_async_copy(emb_ref.at[pl.ds(0, seq)], out_ref, sem)
    copy.start()
    copy.wait()


def kernel(x, emb):
    seq = x.shape[1]
    return pl.pallas_call(
        _copy_body,
        out_shape=jax.ShapeDtypeStruct((seq, emb.shape[1]), emb.dtype),
        in_specs=[pl.BlockSpec(memory_space=pl.ANY)],
        out_specs=pl.BlockSpec(memory_space=pl.ANY),
        scratch_shapes=[pltpu.SemaphoreType.DMA],
    )(emb)


# speedup vs baseline: 1.0029x; 1.0029x over previous
import jax
import jax.numpy as jnp
from jax.experimental import pallas as pl
from jax.experimental.pallas import tpu as pltpu


_NUM_STREAMS = 16


def _copy_body(emb_ref, out_ref, sems):
    seq = out_ref.shape[0]
    chunk = seq // _NUM_STREAMS
    copies = []
    for i in range(_NUM_STREAMS):
        lo = i * chunk
        c = pltpu.make_async_copy(
            emb_ref.at[pl.ds(lo, chunk)], out_ref.at[pl.ds(lo, chunk)],
            sems.at[i])
        c.start()
        copies.append(c)
    for c in copies:
        c.wait()


def kernel(x, emb):
    seq = x.shape[1]
    return pl.pallas_call(
        _copy_body,
        out_shape=jax.ShapeDtypeStruct((seq, emb.shape[1]), emb.dtype),
        in_specs=[pl.BlockSpec(memory_space=pl.ANY)],
        out_specs=pl.BlockSpec(memory_space=pl.ANY),
        scratch_shapes=[pltpu.SemaphoreType.DMA((_NUM_STREAMS,))],
    )(emb)


# speedup vs baseline: 41.7067x; 41.5871x over previous
import jax
import jax.numpy as jnp
from jax.experimental import pallas as pl
from jax.experimental.pallas import tpu as pltpu

_BLK = 512


def _copy_block(emb_ref, o_ref):
    o_ref[...] = emb_ref[...]


def kernel(x, emb):
    seq, dim = x.shape[1], emb.shape[1]
    return pl.pallas_call(
        _copy_block,
        grid=(seq // _BLK,),
        in_specs=[pl.BlockSpec((_BLK, dim), lambda i: (i, 0))],
        out_specs=pl.BlockSpec((_BLK, dim), lambda i: (i, 0)),
        out_shape=jax.ShapeDtypeStruct((seq, dim), emb.dtype),
        compiler_params=pltpu.CompilerParams(
            dimension_semantics=("arbitrary",),
        ),
    )(emb)
